# baseline (speedup 1.0000x reference)
import functools

import jax
import jax.numpy as jnp
from jax import lax
from jax.experimental import pallas as pl
from jax.experimental.pallas import tpu as pltpu
from jax.experimental.pallas import tpu_sc as plsc

LEVELS = 1024
T = 2048
D = 10000
DP = 10048
SIGNAL_MIN = -5.0
SIGNAL_MAX = 5.0
SLICES = [(0, 3), (3, 9), (9, 12), (12, 15), (15, 18), (18, 21), (21, 24),
          (24, 27), (27, 30)]

NC = 2
NS = 16
NW = NC * NS
TPW = T // NW
L8 = 64
CH = DP // L8
ZCH = DP // 32


def _sc_accum_body(eidx_hbm, tidx_hbm, embed_hbm, keys_hbm, time_hbm,
                   lane_hbm, out_hbm, eidx_v, tidx_v, e3, tw, kb, acc,
                   acc2, lane_v, idx4,
                   sem_e0, sem_t0, sem_k0, sem_e1, sem_t1, sem_k1):
    wid = lax.axis_index("s") * NC + lax.axis_index("c")
    base = wid * TPW
    pltpu.sync_copy(eidx_hbm.at[pl.ds(base, TPW)], eidx_v)
    pltpu.sync_copy(tidx_hbm.at[pl.ds(base, TPW)], tidx_v)
    pltpu.sync_copy(lane_hbm, lane_v)

    lv = lane_v[0, :]
    le, lo = plsc.unpack(lv, format=plsc.PackFormat.INTERLEAVED,
                         preferred_element_type=jnp.int16)
    lee, leo = plsc.unpack(le, format=plsc.PackFormat.INTERLEAVED,
                           preferred_element_type=jnp.int32)
    loe, loo = plsc.unpack(lo, format=plsc.PackFormat.INTERLEAVED,
                           preferred_element_type=jnp.int32)
    idx4[0, :] = lee
    idx4[1, :] = leo
    idx4[2, :] = loe
    idx4[3, :] = loo

    def zbody(j, carry):
        acc[0, pl.ds(j * 32, 32)] = jnp.zeros((32,), jnp.int16)
        return carry

    lax.fori_loop(0, ZCH, zbody, 0)

    slots = [(e3.at[0], tw.at[0], kb.at[0], sem_e0, sem_t0, sem_k0),
             (e3.at[1], tw.at[1], kb.at[1], sem_e1, sem_t1, sem_k1)]

    def issue(t, b):
        e3s, tws, kbs, se, st, sk = slots[b]
        pltpu.async_copy(embed_hbm.at[eidx_v.at[t]], e3s, se)
        pltpu.async_copy(time_hbm.at[tidx_v.at[t]], tws, st)
        pltpu.async_copy(keys_hbm.at[pl.ds(base + t, 1)], kbs, sk)

    def wait(t, b):
        e3s, tws, kbs, se, st, sk = slots[b]
        pltpu.make_async_copy(embed_hbm.at[eidx_v.at[t]], e3s, se).wait()
        pltpu.make_async_copy(time_hbm.at[tidx_v.at[t]], tws, st).wait()
        pltpu.make_async_copy(keys_hbm.at[pl.ds(base + t, 1)], kbs, sk).wait()

    def accum(b):
        e3s, tws, kbs, _, _, _ = slots[b]

        def cbody(j, c2):
            s = pl.ds(j * L8, L8)
            e = e3s[0, s] + e3s[1, s] + e3s[2, s]
            p = e * (kbs[0, s] * tws[0, s])
            pe, po = plsc.unpack(p, format=plsc.PackFormat.INTERLEAVED,
                                 preferred_element_type=jnp.int16)
            acc[0, pl.ds(j * L8, 32)] += pe
            acc[0, pl.ds(j * L8 + 32, 32)] += po
            return c2

        lax.fori_loop(0, CH, cbody, 0)

    issue(0, 0)
    issue(1, 1)

    def tbody(i, carry):
        t = 2 * i
        wait(t, 0)
        accum(0)
        issue(t + 2, 0)
        wait(t + 1, 1)
        accum(1)
        issue(t + 3, 1)
        return carry

    lax.fori_loop(0, TPW // 2 - 1, tbody, 0)
    wait(TPW - 2, 0)
    accum(0)
    wait(TPW - 1, 1)
    accum(1)

    i0 = idx4[0, :]
    i1 = idx4[1, :]
    i2 = idx4[2, :]
    i3 = idx4[3, :]

    def wbody(j, carry):
        dbase = j * L8
        he = acc[0, pl.ds(dbase, 32)]
        ho = acc[0, pl.ds(dbase + 32, 32)]
        a, bq = plsc.unpack(he, format=plsc.PackFormat.INTERLEAVED,
                            preferred_element_type=jnp.int32)
        c, dq = plsc.unpack(ho, format=plsc.PackFormat.INTERLEAVED,
                            preferred_element_type=jnp.int32)
        plsc.store_scatter(acc2, [dbase + i0], a.astype(jnp.float32))
        plsc.store_scatter(acc2, [dbase + i1], bq.astype(jnp.float32))
        plsc.store_scatter(acc2, [dbase + i2], c.astype(jnp.float32))
        plsc.store_scatter(acc2, [dbase + i3], dq.astype(jnp.float32))
        return carry

    lax.fori_loop(0, CH, wbody, 0)
    pltpu.sync_copy(acc2.at[pl.ds(0, D)], out_hbm.at[wid])


@functools.lru_cache(maxsize=1)
def _get_sc_accum():
    mesh = plsc.VectorSubcoreMesh(
        core_axis_name="c", subcore_axis_name="s",
        num_cores=NC, num_subcores=NS)
    return pl.kernel(
        _sc_accum_body,
        out_type=jax.ShapeDtypeStruct((NW, D), jnp.float32),
        mesh=mesh,
        scratch_types=[
            pltpu.VMEM((TPW, 3), jnp.int32),
            pltpu.VMEM((TPW, 1), jnp.int32),
            pltpu.VMEM((2, 3, DP), jnp.int8),
            pltpu.VMEM((2, 1, DP), jnp.int8),
            pltpu.VMEM((2, 1, DP), jnp.int8),
            pltpu.VMEM((1, DP), jnp.int16),
            pltpu.VMEM((DP,), jnp.float32),
            pltpu.VMEM((1, L8), jnp.int8),
            pltpu.VMEM((4, 16), jnp.int32),
            pltpu.SemaphoreType.DMA,
            pltpu.SemaphoreType.DMA,
            pltpu.SemaphoreType.DMA,
            pltpu.SemaphoreType.DMA,
            pltpu.SemaphoreType.DMA,
            pltpu.SemaphoreType.DMA,
        ],
        compiler_params=pltpu.CompilerParams(
            use_tc_tiling_on_sc=False, needs_layout_passes=False),
    )


def _tc_combine_body(partial_ref, f_ref, out_ref):
    s = jnp.sum(partial_ref[...], axis=0, keepdims=True)
    v = s * f_ref[...]
    out_ref[...] = jnp.where(v > 0, 1.0, -1.0).astype(jnp.float32)


def _level_idx(x, low, high, num):
    xc = jnp.clip(x, low, high)
    return jnp.round((xc - low) / (high - low) * (num - 1)).astype(jnp.int32)


def kernel(input, feat, embed_w, keys_w, time_w, w0, b0, w1, b1, w2, b2, w3,
           b3, w4, b4, w5, b5, w6, b6, w7, b7, w8, b8):
    eidx = _level_idx(input[:, 1:], SIGNAL_MIN, SIGNAL_MAX, LEVELS)
    tidx = _level_idx(input[:, 0], 0.0, float(T), T).reshape(T, 1)

    pad = ((0, 0), (0, DP - D))
    ei8 = jnp.pad(embed_w.astype(jnp.int8), pad)
    ki8 = jnp.pad(keys_w.astype(jnp.int8), pad)
    ti8 = jnp.pad(time_w.astype(jnp.int8), pad)
    lane = jnp.arange(L8, dtype=jnp.int8).reshape(1, L8)

    partial = _get_sc_accum()(eidx, tidx, ei8, ki8, ti8, lane)

    ws = [w0, w1, w2, w3, w4, w5, w6, w7, w8]
    bs = [b0, b1, b2, b3, b4, b5, b6, b7, b8]
    fs = []
    for i, (lo, hi) in enumerate(SLICES):
        p = feat[lo:hi] @ ws[i].T
        fs.append(jnp.cos(p + bs[i]) * jnp.sin(p))
    ftot = fs[0] * (fs[1] + fs[8]) * (fs[2] + fs[3] + fs[4]) * (
        fs[5] + fs[6] + fs[7])

    out = pl.pallas_call(
        _tc_combine_body,
        out_shape=jax.ShapeDtypeStruct((1, D), jnp.float32),
        in_specs=[
            pl.BlockSpec((NW, D), lambda: (0, 0)),
            pl.BlockSpec((1, D), lambda: (0, 0)),
        ],
        out_specs=pl.BlockSpec((1, D), lambda: (0, 0)),
    )(partial, ftot.reshape(1, D))
    return out.reshape(D)
